# baseline (speedup 1.0000x reference)
import functools

import jax
import jax.numpy as jnp
from jax import lax
from jax.experimental import pallas as pl
from jax.experimental.pallas import tpu as pltpu
from jax.experimental.pallas import tpu_sc as plsc

_NUM_WORKERS = 32
_TV = 512


def _gather_body(bpw, table_hbm, idx_hbm, out_hbm, idx_v, rows_v, sem):
    wid = lax.axis_index("s") * 2 + lax.axis_index("c")
    base = wid * bpw
    pltpu.sync_copy(idx_hbm.at[pl.ds(base, bpw)], idx_v)
    pltpu.async_copy(table_hbm.at[idx_v], rows_v, sem).wait()
    pltpu.sync_copy(rows_v, out_hbm.at[pl.ds(base, bpw)])


def _sc_gather(embed_w, idx_flat):
    t = idx_flat.shape[0]
    hidden = embed_w.shape[1]
    bpw = t // _NUM_WORKERS
    mesh = plsc.VectorSubcoreMesh(core_axis_name="c", subcore_axis_name="s")
    f = pl.kernel(
        functools.partial(_gather_body, bpw),
        mesh=mesh,
        out_type=jax.ShapeDtypeStruct((t, hidden), jnp.float32),
        scratch_types=[
            pltpu.VMEM((bpw,), jnp.int32),
            pltpu.VMEM((bpw, hidden), jnp.float32),
            pltpu.SemaphoreType.DMA,
        ],
    )
    return f(embed_w, idx_flat)


def _mm_body(x_ref, w_ref, o_ref, xn_ref):
    @pl.when(pl.program_id(0) == 0)
    def _():
        x = x_ref[...]
        mu = jnp.mean(x, axis=1, keepdims=True)
        var = jnp.mean((x - mu) ** 2, axis=1, keepdims=True)
        xn_ref[...] = ((x - mu) * lax.rsqrt(var + 1e-5)).astype(jnp.bfloat16)

    o_ref[...] = lax.dot_general(
        xn_ref[...],
        w_ref[...].astype(jnp.bfloat16),
        (((1,), (1,)), ((), ())),
        preferred_element_type=jnp.float32,
    )


def _ln_matmul(x, lm_head_w):
    t, hidden = x.shape
    vocab = lm_head_w.shape[0]
    grid = (pl.cdiv(vocab, _TV),)
    return pl.pallas_call(
        _mm_body,
        grid=grid,
        in_specs=[
            pl.BlockSpec((t, hidden), lambda i: (0, 0)),
            pl.BlockSpec((_TV, hidden), lambda i: (i, 0)),
        ],
        out_specs=pl.BlockSpec((t, _TV), lambda i: (0, i)),
        out_shape=jax.ShapeDtypeStruct((t, vocab), jnp.float32),
        scratch_shapes=[pltpu.VMEM((t, hidden), jnp.bfloat16)],
    )(x, lm_head_w)


def kernel(idx, embed_w, lm_head_w):
    b, t = idx.shape
    vocab, hidden = embed_w.shape
    idx_flat = idx.reshape(b * t)
    x = _sc_gather(embed_w, idx_flat)
    logits = _ln_matmul(x, lm_head_w)
    return logits.reshape(b, t, vocab)
